# baseline (speedup 1.0000x reference)
import jax
import jax.numpy as jnp
from jax.experimental import pallas as pl
from jax.experimental.pallas import tpu as pltpu

_V, _D, _M, _K = 50000, 512, 4, 512
_CHUNK = _D // _M
_TV = 1000
_NBUF = 4


def _dpq_tile_kernel(logits_hbm, cb_ref, out_ref, buf, sem):
    i = pl.program_id(0)
    n = pl.num_programs(0)

    def dma(chunk, slot):
        return pltpu.make_async_copy(
            logits_hbm.at[pl.ds(chunk * _TV, _TV)],
            buf.at[slot],
            sem.at[slot],
        )

    @pl.when(i == 0)
    def _():
        for s in range(_NBUF):
            dma(s, s).start()

    slot = jax.lax.rem(i, _NBUF)
    dma(i, slot).wait()

    x3 = buf[slot]
    for m in range(_M):
        x = x3[:, m, :]
        x = x - jnp.max(x, axis=-1, keepdims=True)
        e = jnp.exp(x)
        attn = e / jnp.sum(e, axis=-1, keepdims=True)
        out_ref[:, m * _CHUNK:(m + 1) * _CHUNK] = jnp.dot(
            attn, cb_ref[m], preferred_element_type=jnp.float32
        )

    @pl.when(i + _NBUF < n)
    def _():
        dma(i + _NBUF, slot).start()


def kernel(assign_logits, codebooks):
    return pl.pallas_call(
        _dpq_tile_kernel,
        grid=(_V // _TV,),
        in_specs=[
            pl.BlockSpec(memory_space=pl.ANY),
            pl.BlockSpec((_M, _K, _CHUNK), lambda i: (0, 0, 0)),
        ],
        out_specs=pl.BlockSpec((_TV, _D), lambda i: (i, 0)),
        out_shape=jax.ShapeDtypeStruct((_V, _D), jnp.float32),
        scratch_shapes=[
            pltpu.VMEM((_NBUF, _TV, _M, _K), jnp.float32),
            pltpu.SemaphoreType.DMA((_NBUF,)),
        ],
    )(assign_logits, codebooks)
